# baseline (speedup 1.0000x reference)
import functools

import jax
import jax.numpy as jnp
from jax import lax
from jax.experimental import pallas as pl
from jax.experimental.pallas import tpu as pltpu
from jax.experimental.pallas import tpu_sc as plsc

EMBED = 64
CHUNK = 512


@functools.cache
def _gather_fn(n_total: int):
    info = plsc.get_sparse_core_info()
    nc, ns = info.num_cores, info.num_subcores
    nw = nc * ns
    per_w = n_total // nw
    n_chunks = per_w // CHUNK
    assert per_w * nw == n_total and n_chunks * CHUNK == per_w
    assert n_chunks % 2 == 0 and n_chunks >= 6

    mesh = plsc.VectorSubcoreMesh(core_axis_name="c", subcore_axis_name="s")

    @functools.partial(
        pl.kernel,
        mesh=mesh,
        out_type=jax.ShapeDtypeStruct((n_total, EMBED), jnp.float32),
        scratch_types=[
            pltpu.VMEM((CHUNK,), jnp.int32),
            pltpu.VMEM((CHUNK,), jnp.int32),
            pltpu.VMEM((CHUNK, EMBED), jnp.float32),
            pltpu.VMEM((CHUNK, EMBED), jnp.float32),
            pltpu.SemaphoreType.DMA,
            pltpu.SemaphoreType.DMA,
            pltpu.SemaphoreType.DMA,
            pltpu.SemaphoreType.DMA,
            pltpu.SemaphoreType.DMA,
            pltpu.SemaphoreType.DMA,
        ],
        compiler_params=pltpu.CompilerParams(use_tc_tiling_on_sc=False),
    )
    def k(idx_hbm, table_hbm, out_hbm, idx0, idx1, rows0, rows1,
          si0, si1, sg0, sg1, so0, so1):
        idx_v = (idx0, idx1)
        rows_v = (rows0, rows1)
        si = (si0, si1)
        sg = (sg0, sg1)
        so = (so0, so1)
        wid = lax.axis_index("s") * nc + lax.axis_index("c")
        w_base = wid * per_w

        def start_idx(i, b):
            pltpu.async_copy(
                idx_hbm.at[pl.ds(w_base + i * CHUNK, CHUNK)], idx_v[b], si[b])

        def wait_idx(b):
            pltpu.make_async_copy(
                idx_hbm.at[pl.ds(0, CHUNK)], idx_v[b], si[b]).wait()

        def start_out(i, b):
            pltpu.async_copy(
                rows_v[b], out_hbm.at[pl.ds(w_base + i * CHUNK, CHUNK)], so[b])

        def wait_out(b):
            pltpu.make_async_copy(
                rows_v[b], out_hbm.at[pl.ds(0, CHUNK)], so[b]).wait()

        def step(i, b, *, first, prefetch):
            if not first:
                wait_out(b)
            wait_idx(b)
            cp = pltpu.async_copy(table_hbm.at[idx_v[b]], rows_v[b], sg[b])
            if prefetch:
                start_idx(i + 1, 1 - b)
            cp.wait()
            start_out(i, b)

        start_idx(0, 0)
        step(0, 0, first=True, prefetch=True)
        step(1, 1, first=True, prefetch=True)

        def outer(j, carry):
            step2 = j * 2
            step(step2, 0, first=False, prefetch=True)
            step(step2 + 1, 1, first=False, prefetch=True)
            return carry

        lax.fori_loop(1, n_chunks // 2 - 1, outer, 0)

        step(n_chunks - 2, 0, first=False, prefetch=True)
        step(n_chunks - 1, 1, first=False, prefetch=False)
        wait_out(0)
        wait_out(1)

    return k


def kernel(sentences, table):
    b, s = sentences.shape
    n = b * s
    idx = sentences.reshape(n).astype(jnp.int32)
    out = _gather_fn(n)(idx, table)
    return out.reshape(b, s, EMBED)


# speedup vs baseline: 1.0025x; 1.0025x over previous
import functools

import jax
import jax.numpy as jnp
from jax import lax
from jax.experimental import pallas as pl
from jax.experimental.pallas import tpu as pltpu
from jax.experimental.pallas import tpu_sc as plsc

EMBED = 64
CHUNK = 512


@functools.cache
def _gather_fn(n_total: int):
    info = plsc.get_sparse_core_info()
    nc, ns = info.num_cores, info.num_subcores
    nw = nc * ns
    per_w = n_total // nw
    n_chunks = per_w // CHUNK
    assert per_w * nw == n_total and n_chunks * CHUNK == per_w
    assert n_chunks % 2 == 0 and n_chunks >= 6

    mesh = plsc.VectorSubcoreMesh(core_axis_name="c", subcore_axis_name="s")

    @functools.partial(
        pl.kernel,
        mesh=mesh,
        out_type=jax.ShapeDtypeStruct((n_total, EMBED), jnp.float32),
        scratch_types=[
            pltpu.VMEM((CHUNK,), jnp.int32),
            pltpu.VMEM((CHUNK,), jnp.int32),
            pltpu.VMEM((CHUNK, EMBED), jnp.float32),
            pltpu.VMEM((CHUNK, EMBED), jnp.float32),
            pltpu.SemaphoreType.DMA,
            pltpu.SemaphoreType.DMA,
            pltpu.SemaphoreType.DMA,
            pltpu.SemaphoreType.DMA,
            pltpu.SemaphoreType.DMA,
            pltpu.SemaphoreType.DMA,
        ],
        compiler_params=pltpu.CompilerParams(use_tc_tiling_on_sc=False),
    )
    def k(idx_hbm, table_hbm, out_hbm, idx0, idx1, rows0, rows1,
          si0, si1, sg0, sg1, so0, so1):
        idx_v = (idx0, idx1)
        rows_v = (rows0, rows1)
        si = (si0, si1)
        sg = (sg0, sg1)
        so = (so0, so1)
        wid = lax.axis_index("s") * nc + lax.axis_index("c")
        w_base = wid * per_w

        def start_idx(i, b):
            pltpu.async_copy(
                idx_hbm.at[pl.ds(w_base + i * CHUNK, CHUNK)], idx_v[b], si[b])

        def wait_idx(b):
            pltpu.make_async_copy(
                idx_hbm.at[pl.ds(0, CHUNK)], idx_v[b], si[b]).wait()

        def start_out(i, b):
            pltpu.async_copy(
                rows_v[b], out_hbm.at[pl.ds(w_base + i * CHUNK, CHUNK)], so[b])

        def wait_out(b):
            pltpu.make_async_copy(
                rows_v[b], out_hbm.at[pl.ds(0, CHUNK)], so[b]).wait()

        def wait_gather(b):
            pltpu.make_async_copy(
                table_hbm.at[idx_v[b]], rows_v[b], sg[b]).wait()

        def step(i, b, *, slot_wait, drain_prev, prefetch):
            if slot_wait:
                wait_out(b)
            wait_idx(b)
            pltpu.async_copy(table_hbm.at[idx_v[b]], rows_v[b], sg[b])
            if drain_prev:
                wait_gather(1 - b)
                start_out(i - 1, 1 - b)
            if prefetch:
                start_idx(i + 1, 1 - b)

        start_idx(0, 0)
        step(0, 0, slot_wait=False, drain_prev=False, prefetch=True)
        step(1, 1, slot_wait=False, drain_prev=True, prefetch=True)

        def outer(j, carry):
            i2 = j * 2
            step(i2, 0, slot_wait=True, drain_prev=True, prefetch=True)
            step(i2 + 1, 1, slot_wait=True, drain_prev=True, prefetch=True)
            return carry

        lax.fori_loop(1, n_chunks // 2 - 1, outer, 0)

        step(n_chunks - 2, 0, slot_wait=True, drain_prev=True, prefetch=True)
        step(n_chunks - 1, 1, slot_wait=True, drain_prev=True, prefetch=False)
        wait_gather(1)
        start_out(n_chunks - 1, 1)
        wait_out(0)
        wait_out(1)

    return k


def kernel(sentences, table):
    b, s = sentences.shape
    n = b * s
    idx = sentences.reshape(n).astype(jnp.int32)
    out = _gather_fn(n)(idx, table)
    return out.reshape(b, s, EMBED)


# speedup vs baseline: 1.0314x; 1.0288x over previous
import functools

import jax
import jax.numpy as jnp
from jax import lax
from jax.experimental import pallas as pl
from jax.experimental.pallas import tpu as pltpu
from jax.experimental.pallas import tpu_sc as plsc

EMBED = 64
CHUNK = 512


@functools.cache
def _gather_fn(n_total: int):
    info = plsc.get_sparse_core_info()
    nc, ns = info.num_cores, info.num_subcores
    nw = nc * ns
    per_w = n_total // nw
    n_chunks = per_w // CHUNK
    assert per_w * nw == n_total and n_chunks * CHUNK == per_w
    assert n_chunks % 2 == 0 and n_chunks >= 6

    mesh = plsc.VectorSubcoreMesh(core_axis_name="c", subcore_axis_name="s")

    @functools.partial(
        pl.kernel,
        mesh=mesh,
        out_type=jax.ShapeDtypeStruct((n_total, EMBED), jnp.float32),
        scratch_types=[
            pltpu.VMEM((CHUNK,), jnp.int32),
            pltpu.VMEM((CHUNK,), jnp.int32),
            pltpu.VMEM((CHUNK, EMBED), jnp.float32),
            pltpu.VMEM((CHUNK, EMBED), jnp.float32),
            pltpu.SemaphoreType.DMA,
            pltpu.SemaphoreType.DMA,
            pltpu.SemaphoreType.DMA,
            pltpu.SemaphoreType.DMA,
            pltpu.SemaphoreType.DMA,
            pltpu.SemaphoreType.DMA,
        ],
        compiler_params=pltpu.CompilerParams(use_tc_tiling_on_sc=False),
    )
    def k(idx_hbm, table_hbm, out_hbm, idx0, idx1, rows0, rows1,
          si0, si1, sg0, sg1, so0, so1):
        idx_v = (idx0, idx1)
        rows_v = (rows0, rows1)
        si = (si0, si1)
        sg = (sg0, sg1)
        so = (so0, so1)
        wid = lax.axis_index("s") * nc + lax.axis_index("c")
        w_base = wid * per_w

        def start_idx(i, b):
            pltpu.async_copy(
                idx_hbm.at[pl.ds(w_base + i * CHUNK, CHUNK)], idx_v[b], si[b])

        def wait_idx(b):
            pltpu.make_async_copy(
                idx_hbm.at[pl.ds(0, CHUNK)], idx_v[b], si[b]).wait()

        def start_out(i, b):
            pltpu.async_copy(
                rows_v[b], out_hbm.at[pl.ds(w_base + i * CHUNK, CHUNK)], so[b])

        def wait_out(b):
            pltpu.make_async_copy(
                rows_v[b], out_hbm.at[pl.ds(0, CHUNK)], so[b]).wait()

        def wait_gather(b):
            pltpu.make_async_copy(
                table_hbm.at[idx_v[b]], rows_v[b], sg[b]).wait()

        def step(i, b, *, slot_wait, drain_prev, prefetch):
            if slot_wait:
                wait_out(b)
            wait_idx(b)
            pltpu.async_copy(table_hbm.at[idx_v[b]], rows_v[b], sg[b])
            if drain_prev:
                wait_gather(1 - b)
                start_out(i - 1, 1 - b)
            if prefetch:
                start_idx(i + 1, 1 - b)

        start_idx(0, 0)
        step(0, 0, slot_wait=False, drain_prev=False, prefetch=True)
        step(1, 1, slot_wait=False, drain_prev=True, prefetch=True)

        def outer(j, carry):
            i2 = j * 2
            step(i2, 0, slot_wait=True, drain_prev=True, prefetch=True)
            step(i2 + 1, 1, slot_wait=True, drain_prev=True, prefetch=True)
            return carry

        lax.fori_loop(1, n_chunks // 2 - 1, outer, 0)

        step(n_chunks - 2, 0, slot_wait=True, drain_prev=True, prefetch=True)
        step(n_chunks - 1, 1, slot_wait=True, drain_prev=True, prefetch=False)
        wait_gather(1)
        start_out(n_chunks - 1, 1)
        wait_out(0)
        wait_out(1)

    return k


def kernel(sentences, table):
    b, s = sentences.shape
    n = b * s
    idx = sentences.T.reshape(n).astype(jnp.int32)
    out = _gather_fn(n)(idx, table)
    return out.reshape(s, b, EMBED).transpose(1, 0, 2)
